# Initial kernel scaffold; baseline (speedup 1.0000x reference)
import jax
import jax.numpy as jnp
from jax.experimental import pallas as pl


def kernel(x, edge_index, Wl1, bl1, Wr1, Wl2, bl2, Wr2):
    raise NotImplementedError("write your pallas kernel here")


# speedup vs baseline: 10.7309x; 10.7309x over previous
import functools

import jax
import jax.numpy as jnp
from jax import lax
from jax.experimental import pallas as pl
from jax.experimental.pallas import tpu as pltpu
from jax.experimental.pallas import tpu_sc as plsc

N_CORES = 2
N_SUBCORES = 16
N_TILES = N_CORES * N_SUBCORES
LANES = 16
CHUNK = 128
BM = 512


def _proj_body(x_ref, wl_ref, wr_ref, y_ref, xr_ref):
    x = x_ref[...]
    y_ref[...] = jnp.dot(x, wl_ref[...], preferred_element_type=jnp.float32)
    xr_ref[...] = jnp.dot(x, wr_ref[...], preferred_element_type=jnp.float32)


def _mid_body(acc_ref, cnt_ref, xr_ref, bl_ref, wl2_ref, wr2_ref, y2_ref, hr_ref):
    agg = acc_ref[0] + acc_ref[1]
    cnt = cnt_ref[0, :, 0:1] + cnt_ref[1, :, 0:1]
    agg = agg / jnp.maximum(cnt, 1.0)
    pre = agg + bl_ref[...] + xr_ref[...]
    norm = jnp.sqrt(jnp.sum(pre * pre, axis=1, keepdims=True))
    h = pre / jnp.maximum(norm, 1e-12)
    h = jnp.maximum(h, 0.0)
    y2_ref[...] = jnp.dot(h, wl2_ref[...], preferred_element_type=jnp.float32)
    hr_ref[...] = jnp.dot(h, wr2_ref[...], preferred_element_type=jnp.float32)


def _out_body(acc_ref, cnt_ref, hr_ref, bl_ref, out_ref):
    agg = acc_ref[0] + acc_ref[1]
    cnt = cnt_ref[0, :, 0:1] + cnt_ref[1, :, 0:1]
    agg = agg / jnp.maximum(cnt, 1.0)
    pre = agg + bl_ref[...] + hr_ref[...]
    norm = jnp.sqrt(jnp.sum(pre * pre, axis=1, keepdims=True))
    o = pre / jnp.maximum(norm, 1e-12)
    z = o - jnp.max(o, axis=1, keepdims=True)
    out_ref[...] = z - jnp.log(jnp.sum(jnp.exp(z), axis=1, keepdims=True))


def _make_sc_segment_sum(n_pad, width, n_chunks, with_count):
    rows_per_tile = n_pad // N_SUBCORES
    mesh = plsc.VectorSubcoreMesh(core_axis_name="c", subcore_axis_name="s")

    out_type = [jax.ShapeDtypeStruct((N_CORES, n_pad, width), jnp.float32)]
    scratch = [
        pltpu.VMEM((n_chunks, CHUNK), jnp.int32),
        pltpu.VMEM((n_chunks, CHUNK), jnp.int32),
        pltpu.VMEM((CHUNK, width), jnp.float32),
        pltpu.VMEM_SHARED((n_pad, width), jnp.float32),
        pltpu.SemaphoreType.DMA,
    ]
    if with_count:
        out_type.append(jax.ShapeDtypeStruct((N_CORES, n_pad, LANES), jnp.float32))
        scratch += [
            pltpu.VMEM((CHUNK, LANES), jnp.float32),
            pltpu.VMEM_SHARED((n_pad, LANES), jnp.float32),
        ]

    def body(*refs):
        if with_count:
            (src_hbm, dst_hbm, tbl_hbm, z_hbm, zc_hbm, ones_hbm,
             acc_out, cnt_out,
             src_v, dst_v, msg_v, acc_sh, sem, ones_v, cnt_sh) = refs
        else:
            (src_hbm, dst_hbm, tbl_hbm, z_hbm,
             acc_out,
             src_v, dst_v, msg_v, acc_sh, sem) = refs
        c = lax.axis_index("c")
        s = lax.axis_index("s")
        wid = c * N_SUBCORES + s
        r0 = s * rows_per_tile
        rows = pl.ds(r0, rows_per_tile)
        pltpu.sync_copy(z_hbm.at[rows], acc_sh.at[rows])
        if with_count:
            pltpu.sync_copy(zc_hbm.at[rows], cnt_sh.at[rows])
            pltpu.sync_copy(ones_hbm, ones_v)
        pltpu.sync_copy(src_hbm.at[wid], src_v)
        pltpu.sync_copy(dst_hbm.at[wid], dst_v)
        plsc.subcore_barrier()

        def step(j, carry):
            pltpu.async_copy(tbl_hbm.at[src_v.at[j]], msg_v, sem).wait()
            pltpu.sync_copy(msg_v, acc_sh.at[dst_v.at[j]], add=True)
            if with_count:
                pltpu.sync_copy(ones_v, cnt_sh.at[dst_v.at[j]], add=True)
            return carry

        lax.fori_loop(0, n_chunks, step, 0)
        plsc.subcore_barrier()
        pltpu.sync_copy(acc_sh.at[rows], acc_out.at[c, rows])
        if with_count:
            pltpu.sync_copy(cnt_sh.at[rows], cnt_out.at[c, rows])

    return pl.kernel(
        body, out_type=out_type, mesh=mesh, scratch_types=scratch,
        compiler_params=pltpu.CompilerParams(use_tc_tiling_on_sc=False))


def _tc_proj(x_pad, Wl, Wr, n_pad, in_ch, out_w):
    grid = (n_pad // BM,)
    return pl.pallas_call(
        _proj_body,
        grid=grid,
        in_specs=[
            pl.BlockSpec((BM, in_ch), lambda i: (i, 0)),
            pl.BlockSpec((in_ch, out_w), lambda i: (0, 0)),
            pl.BlockSpec((in_ch, out_w), lambda i: (0, 0)),
        ],
        out_specs=[
            pl.BlockSpec((BM, out_w), lambda i: (i, 0)),
            pl.BlockSpec((BM, out_w), lambda i: (i, 0)),
        ],
        out_shape=[
            jax.ShapeDtypeStruct((n_pad, out_w), jnp.float32),
            jax.ShapeDtypeStruct((n_pad, out_w), jnp.float32),
        ],
    )(x_pad, Wl, Wr)


def _tc_mid(acc, cnt, xr, bl, Wl2, Wr2, n_pad, hid, out_ch):
    grid = (n_pad // BM,)
    return pl.pallas_call(
        _mid_body,
        grid=grid,
        in_specs=[
            pl.BlockSpec((N_CORES, BM, hid), lambda i: (0, i, 0)),
            pl.BlockSpec((N_CORES, BM, LANES), lambda i: (0, i, 0)),
            pl.BlockSpec((BM, hid), lambda i: (i, 0)),
            pl.BlockSpec((1, hid), lambda i: (0, 0)),
            pl.BlockSpec((hid, out_ch), lambda i: (0, 0)),
            pl.BlockSpec((hid, out_ch), lambda i: (0, 0)),
        ],
        out_specs=[
            pl.BlockSpec((BM, out_ch), lambda i: (i, 0)),
            pl.BlockSpec((BM, out_ch), lambda i: (i, 0)),
        ],
        out_shape=[
            jax.ShapeDtypeStruct((n_pad, out_ch), jnp.float32),
            jax.ShapeDtypeStruct((n_pad, out_ch), jnp.float32),
        ],
    )(acc, cnt, xr, bl, Wl2, Wr2)


def _tc_out(acc, cnt, hr, bl, n_pad, out_ch):
    grid = (n_pad // BM,)
    return pl.pallas_call(
        _out_body,
        grid=grid,
        in_specs=[
            pl.BlockSpec((N_CORES, BM, out_ch), lambda i: (0, i, 0)),
            pl.BlockSpec((N_CORES, BM, LANES), lambda i: (0, i, 0)),
            pl.BlockSpec((BM, out_ch), lambda i: (i, 0)),
            pl.BlockSpec((1, out_ch), lambda i: (0, 0)),
        ],
        out_specs=pl.BlockSpec((BM, out_ch), lambda i: (i, 0)),
        out_shape=jax.ShapeDtypeStruct((n_pad, out_ch), jnp.float32),
    )(acc, cnt, hr, bl)


def kernel(x, edge_index, Wl1, bl1, Wr1, Wl2, bl2, Wr2):
    n, in_ch = x.shape
    hid = Wl1.shape[1]
    out_ch = Wl2.shape[1]
    e = edge_index.shape[1]

    n_pad = -(-n // (N_SUBCORES * BM // 8)) * (N_SUBCORES * BM // 8)
    n_pad = max(n_pad, N_SUBCORES * 8)
    per_tile = -(-e // (N_TILES * CHUNK)) * CHUNK
    n_chunks = per_tile // CHUNK
    e_pad = N_TILES * per_tile

    src = edge_index[0].astype(jnp.int32)
    dst = edge_index[1].astype(jnp.int32)
    pad_node = jnp.int32(n_pad - 1)
    src = jnp.concatenate([src, jnp.full((e_pad - e,), pad_node)])
    dst = jnp.concatenate([dst, jnp.full((e_pad - e,), pad_node)])
    src = src.reshape(N_TILES, n_chunks, CHUNK)
    dst = dst.reshape(N_TILES, n_chunks, CHUNK)

    x_pad = jnp.pad(x, ((0, n_pad - n), (0, 0)))

    zeros_hid = jnp.zeros((n_pad, hid), jnp.float32)
    zeros_cnt = jnp.zeros((n_pad, LANES), jnp.float32)
    zeros_out = jnp.zeros((n_pad, out_ch), jnp.float32)
    ones_rows = jnp.ones((CHUNK, LANES), jnp.float32)

    y1, xr1 = _tc_proj(x_pad, Wl1, Wr1, n_pad, in_ch, hid)
    sc1 = _make_sc_segment_sum(n_pad, hid, n_chunks, with_count=True)
    acc1, cnt = sc1(src, dst, y1, zeros_hid, zeros_cnt, ones_rows)
    y2, hr2 = _tc_mid(acc1, cnt, xr1, bl1.reshape(1, hid), Wl2, Wr2,
                      n_pad, hid, out_ch)

    sc2 = _make_sc_segment_sum(n_pad, out_ch, n_chunks, with_count=False)
    (acc2,) = sc2(src, dst, y2, zeros_out)
    out = _tc_out(acc2, cnt, hr2, bl2.reshape(1, out_ch), n_pad, out_ch)
    return out[:n]


# speedup vs baseline: 15.1333x; 1.4103x over previous
import functools

import jax
import jax.numpy as jnp
from jax import lax
from jax.experimental import pallas as pl
from jax.experimental.pallas import tpu as pltpu
from jax.experimental.pallas import tpu_sc as plsc

N_CORES = 2
N_SUBCORES = 16
N_TILES = N_CORES * N_SUBCORES
LANES = 16
CHUNK = 128
BM = 512


def _proj_body(x_ref, wl_ref, wr_ref, y_ref, xr_ref):
    x = x_ref[...]
    y_ref[...] = jnp.dot(x, wl_ref[...], preferred_element_type=jnp.float32)
    xr_ref[...] = jnp.dot(x, wr_ref[...], preferred_element_type=jnp.float32)


def _mid_body(acc_ref, cnt_ref, xr_ref, bl_ref, wl2_ref, wr2_ref, y2_ref, hr_ref):
    agg = acc_ref[0] + acc_ref[1]
    cnt = cnt_ref[0, :, 0:1] + cnt_ref[1, :, 0:1]
    agg = agg / jnp.maximum(cnt, 1.0)
    pre = agg + bl_ref[...] + xr_ref[...]
    norm = jnp.sqrt(jnp.sum(pre * pre, axis=1, keepdims=True))
    h = pre / jnp.maximum(norm, 1e-12)
    h = jnp.maximum(h, 0.0)
    y2_ref[...] = jnp.dot(h, wl2_ref[...], preferred_element_type=jnp.float32)
    hr_ref[...] = jnp.dot(h, wr2_ref[...], preferred_element_type=jnp.float32)


def _out_body(acc_ref, cnt_ref, hr_ref, bl_ref, out_ref):
    agg = acc_ref[0] + acc_ref[1]
    cnt = cnt_ref[0, :, 0:1] + cnt_ref[1, :, 0:1]
    agg = agg / jnp.maximum(cnt, 1.0)
    pre = agg + bl_ref[...] + hr_ref[...]
    norm = jnp.sqrt(jnp.sum(pre * pre, axis=1, keepdims=True))
    o = pre / jnp.maximum(norm, 1e-12)
    z = o - jnp.max(o, axis=1, keepdims=True)
    out_ref[...] = z - jnp.log(jnp.sum(jnp.exp(z), axis=1, keepdims=True))


def _make_sc_segment_sum(n_pad, width, n_chunks, with_count):
    rows_per_tile = n_pad // N_SUBCORES
    mesh = plsc.VectorSubcoreMesh(core_axis_name="c", subcore_axis_name="s")
    nb = min(4, n_chunks)

    out_type = [jax.ShapeDtypeStruct((N_CORES, n_pad, width), jnp.float32)]
    scratch = [
        pltpu.VMEM((n_chunks, CHUNK), jnp.int32),
        pltpu.VMEM((n_chunks, CHUNK), jnp.int32),
        pltpu.VMEM((nb, CHUNK, width), jnp.float32),
        pltpu.VMEM_SHARED((n_pad, width), jnp.float32),
        pltpu.SemaphoreType.DMA((nb,)),
        pltpu.SemaphoreType.DMA((nb,)),
    ]
    if with_count:
        out_type.append(jax.ShapeDtypeStruct((N_CORES, n_pad, LANES), jnp.float32))
        scratch += [
            pltpu.VMEM((CHUNK, LANES), jnp.float32),
            pltpu.VMEM_SHARED((n_pad, LANES), jnp.float32),
            pltpu.SemaphoreType.DMA((nb,)),
        ]

    def body(*refs):
        if with_count:
            (src_hbm, dst_hbm, tbl_hbm, z_hbm, zc_hbm, ones_hbm,
             acc_out, cnt_out,
             src_v, dst_v, msg_v, acc_sh, gsem, ssem, ones_v, cnt_sh, csem) = refs
        else:
            (src_hbm, dst_hbm, tbl_hbm, z_hbm,
             acc_out,
             src_v, dst_v, msg_v, acc_sh, gsem, ssem) = refs
        c = lax.axis_index("c")
        s = lax.axis_index("s")
        wid = c * N_SUBCORES + s
        r0 = s * rows_per_tile
        rows = pl.ds(r0, rows_per_tile)
        pltpu.sync_copy(z_hbm.at[rows], acc_sh.at[rows])
        if with_count:
            pltpu.sync_copy(zc_hbm.at[rows], cnt_sh.at[rows])
            pltpu.sync_copy(ones_hbm, ones_v)
        pltpu.sync_copy(src_hbm.at[wid], src_v)
        pltpu.sync_copy(dst_hbm.at[wid], dst_v)
        plsc.subcore_barrier()

        def gather(j, b):
            pltpu.async_copy(tbl_hbm.at[src_v.at[j]], msg_v.at[b], gsem.at[b])

        for b in range(nb):
            gather(b, b)

        def step(j, carry):
            b = lax.rem(j, nb)
            if with_count:
                @pl.when(j >= nb)
                def _():
                    pltpu.make_async_copy(
                        ones_v, cnt_sh.at[dst_v.at[j]], csem.at[b]).wait()
            pltpu.make_async_copy(
                tbl_hbm.at[src_v.at[j]], msg_v.at[b], gsem.at[b]).wait()
            pltpu.async_copy(msg_v.at[b], acc_sh.at[dst_v.at[j]], ssem.at[b],
                             add=True)
            if with_count:
                pltpu.async_copy(ones_v, cnt_sh.at[dst_v.at[j]], csem.at[b],
                                 add=True)

            @pl.when(j + nb < n_chunks)
            def _():
                pltpu.make_async_copy(
                    msg_v.at[b], acc_sh.at[dst_v.at[j]], ssem.at[b]).wait()
                gather(j + nb, b)

            return carry

        lax.fori_loop(0, n_chunks, step, 0)
        for b in range(nb):
            pltpu.make_async_copy(
                msg_v.at[b], acc_sh.at[dst_v.at[0]], ssem.at[b]).wait()
            if with_count:
                pltpu.make_async_copy(
                    ones_v, cnt_sh.at[dst_v.at[0]], csem.at[b]).wait()
        plsc.subcore_barrier()
        pltpu.sync_copy(acc_sh.at[rows], acc_out.at[c, rows])
        if with_count:
            pltpu.sync_copy(cnt_sh.at[rows], cnt_out.at[c, rows])

    return pl.kernel(
        body, out_type=out_type, mesh=mesh, scratch_types=scratch,
        compiler_params=pltpu.CompilerParams(use_tc_tiling_on_sc=False))


def _tc_proj(x_pad, Wl, Wr, n_pad, in_ch, out_w):
    grid = (n_pad // BM,)
    return pl.pallas_call(
        _proj_body,
        grid=grid,
        in_specs=[
            pl.BlockSpec((BM, in_ch), lambda i: (i, 0)),
            pl.BlockSpec((in_ch, out_w), lambda i: (0, 0)),
            pl.BlockSpec((in_ch, out_w), lambda i: (0, 0)),
        ],
        out_specs=[
            pl.BlockSpec((BM, out_w), lambda i: (i, 0)),
            pl.BlockSpec((BM, out_w), lambda i: (i, 0)),
        ],
        out_shape=[
            jax.ShapeDtypeStruct((n_pad, out_w), jnp.float32),
            jax.ShapeDtypeStruct((n_pad, out_w), jnp.float32),
        ],
    )(x_pad, Wl, Wr)


def _tc_mid(acc, cnt, xr, bl, Wl2, Wr2, n_pad, hid, out_ch):
    grid = (n_pad // BM,)
    return pl.pallas_call(
        _mid_body,
        grid=grid,
        in_specs=[
            pl.BlockSpec((N_CORES, BM, hid), lambda i: (0, i, 0)),
            pl.BlockSpec((N_CORES, BM, LANES), lambda i: (0, i, 0)),
            pl.BlockSpec((BM, hid), lambda i: (i, 0)),
            pl.BlockSpec((1, hid), lambda i: (0, 0)),
            pl.BlockSpec((hid, out_ch), lambda i: (0, 0)),
            pl.BlockSpec((hid, out_ch), lambda i: (0, 0)),
        ],
        out_specs=[
            pl.BlockSpec((BM, out_ch), lambda i: (i, 0)),
            pl.BlockSpec((BM, out_ch), lambda i: (i, 0)),
        ],
        out_shape=[
            jax.ShapeDtypeStruct((n_pad, out_ch), jnp.float32),
            jax.ShapeDtypeStruct((n_pad, out_ch), jnp.float32),
        ],
    )(acc, cnt, xr, bl, Wl2, Wr2)


def _tc_out(acc, cnt, hr, bl, n_pad, out_ch):
    grid = (n_pad // BM,)
    return pl.pallas_call(
        _out_body,
        grid=grid,
        in_specs=[
            pl.BlockSpec((N_CORES, BM, out_ch), lambda i: (0, i, 0)),
            pl.BlockSpec((N_CORES, BM, LANES), lambda i: (0, i, 0)),
            pl.BlockSpec((BM, out_ch), lambda i: (i, 0)),
            pl.BlockSpec((1, out_ch), lambda i: (0, 0)),
        ],
        out_specs=pl.BlockSpec((BM, out_ch), lambda i: (i, 0)),
        out_shape=jax.ShapeDtypeStruct((n_pad, out_ch), jnp.float32),
    )(acc, cnt, hr, bl)


def kernel(x, edge_index, Wl1, bl1, Wr1, Wl2, bl2, Wr2):
    n, in_ch = x.shape
    hid = Wl1.shape[1]
    out_ch = Wl2.shape[1]
    e = edge_index.shape[1]

    n_pad = -(-n // (N_SUBCORES * BM // 8)) * (N_SUBCORES * BM // 8)
    n_pad = max(n_pad, N_SUBCORES * 8)
    per_tile = -(-e // (N_TILES * CHUNK)) * CHUNK
    n_chunks = per_tile // CHUNK
    e_pad = N_TILES * per_tile

    src = edge_index[0].astype(jnp.int32)
    dst = edge_index[1].astype(jnp.int32)
    pad_node = jnp.int32(n_pad - 1)
    src = jnp.concatenate([src, jnp.full((e_pad - e,), pad_node)])
    dst = jnp.concatenate([dst, jnp.full((e_pad - e,), pad_node)])
    src = src.reshape(N_TILES, n_chunks, CHUNK)
    dst = dst.reshape(N_TILES, n_chunks, CHUNK)

    x_pad = jnp.pad(x, ((0, n_pad - n), (0, 0)))

    zeros_hid = jnp.zeros((n_pad, hid), jnp.float32)
    zeros_cnt = jnp.zeros((n_pad, LANES), jnp.float32)
    zeros_out = jnp.zeros((n_pad, out_ch), jnp.float32)
    ones_rows = jnp.ones((CHUNK, LANES), jnp.float32)

    y1, xr1 = _tc_proj(x_pad, Wl1, Wr1, n_pad, in_ch, hid)
    sc1 = _make_sc_segment_sum(n_pad, hid, n_chunks, with_count=True)
    acc1, cnt = sc1(src, dst, y1, zeros_hid, zeros_cnt, ones_rows)
    y2, hr2 = _tc_mid(acc1, cnt, xr1, bl1.reshape(1, hid), Wl2, Wr2,
                      n_pad, hid, out_ch)

    sc2 = _make_sc_segment_sum(n_pad, out_ch, n_chunks, with_count=False)
    (acc2,) = sc2(src, dst, y2, zeros_out)
    out = _tc_out(acc2, cnt, hr2, bl2.reshape(1, out_ch), n_pad, out_ch)
    return out[:n]


# speedup vs baseline: 15.4419x; 1.0204x over previous
import functools

import jax
import jax.numpy as jnp
from jax import lax
from jax.experimental import pallas as pl
from jax.experimental.pallas import tpu as pltpu
from jax.experimental.pallas import tpu_sc as plsc

N_CORES = 2
N_SUBCORES = 16
N_TILES = N_CORES * N_SUBCORES
LANES = 16
CHUNK = 128
BM = 512


def _proj_body(x_ref, wl_ref, wr_ref, y_ref, xr_ref):
    x = x_ref[...]
    y_ref[...] = jnp.dot(x, wl_ref[...], preferred_element_type=jnp.float32)
    xr_ref[...] = jnp.dot(x, wr_ref[...], preferred_element_type=jnp.float32)


def _mid_body(acc_ref, cnt_ref, xr_ref, bl_ref, wl2_ref, wr2_ref, y2_ref, hr_ref):
    agg = acc_ref[0] + acc_ref[1]
    cnt = cnt_ref[0, :, 0:1] + cnt_ref[1, :, 0:1]
    agg = agg / jnp.maximum(cnt, 1.0)
    pre = agg + bl_ref[...] + xr_ref[...]
    norm = jnp.sqrt(jnp.sum(pre * pre, axis=1, keepdims=True))
    h = pre / jnp.maximum(norm, 1e-12)
    h = jnp.maximum(h, 0.0)
    y2_ref[...] = jnp.dot(h, wl2_ref[...], preferred_element_type=jnp.float32)
    hr_ref[...] = jnp.dot(h, wr2_ref[...], preferred_element_type=jnp.float32)


def _out_body(acc_ref, cnt_ref, hr_ref, bl_ref, out_ref):
    agg = acc_ref[0] + acc_ref[1]
    cnt = cnt_ref[0, :, 0:1] + cnt_ref[1, :, 0:1]
    agg = agg / jnp.maximum(cnt, 1.0)
    pre = agg + bl_ref[...] + hr_ref[...]
    norm = jnp.sqrt(jnp.sum(pre * pre, axis=1, keepdims=True))
    o = pre / jnp.maximum(norm, 1e-12)
    z = o - jnp.max(o, axis=1, keepdims=True)
    out_ref[...] = z - jnp.log(jnp.sum(jnp.exp(z), axis=1, keepdims=True))


def _make_sc_segment_sum(n_pad, width, n_chunks, with_count):
    rows_per_tile = n_pad // N_SUBCORES
    mesh = plsc.VectorSubcoreMesh(core_axis_name="c", subcore_axis_name="s")
    nb = min(8, n_chunks)

    out_type = [jax.ShapeDtypeStruct((N_CORES, n_pad, width), jnp.float32)]
    scratch = [
        pltpu.VMEM((n_chunks, CHUNK), jnp.int32),
        pltpu.VMEM((n_chunks, CHUNK), jnp.int32),
        pltpu.VMEM((nb, CHUNK, width), jnp.float32),
        pltpu.VMEM_SHARED((n_pad, width), jnp.float32),
        pltpu.SemaphoreType.DMA((nb,)),
        pltpu.SemaphoreType.DMA((nb,)),
    ]
    if with_count:
        out_type.append(jax.ShapeDtypeStruct((N_CORES, n_pad, LANES), jnp.float32))
        scratch += [
            pltpu.VMEM((CHUNK, LANES), jnp.float32),
            pltpu.VMEM_SHARED((n_pad, LANES), jnp.float32),
            pltpu.SemaphoreType.DMA((nb,)),
        ]

    def body(*refs):
        if with_count:
            (src_hbm, dst_hbm, tbl_hbm, z_hbm, zc_hbm, ones_hbm,
             acc_out, cnt_out,
             src_v, dst_v, msg_v, acc_sh, gsem, ssem, ones_v, cnt_sh, csem) = refs
        else:
            (src_hbm, dst_hbm, tbl_hbm, z_hbm,
             acc_out,
             src_v, dst_v, msg_v, acc_sh, gsem, ssem) = refs
        c = lax.axis_index("c")
        s = lax.axis_index("s")
        wid = c * N_SUBCORES + s
        r0 = s * rows_per_tile
        rows = pl.ds(r0, rows_per_tile)
        pltpu.sync_copy(z_hbm.at[rows], acc_sh.at[rows])
        if with_count:
            pltpu.sync_copy(zc_hbm.at[rows], cnt_sh.at[rows])
            pltpu.sync_copy(ones_hbm, ones_v)
        pltpu.sync_copy(src_hbm.at[wid], src_v)
        pltpu.sync_copy(dst_hbm.at[wid], dst_v)
        plsc.subcore_barrier()

        def gather(j, b):
            pltpu.async_copy(tbl_hbm.at[src_v.at[j]], msg_v.at[b], gsem.at[b])

        for b in range(nb):
            gather(b, b)

        def step(j, carry):
            b = lax.rem(j, nb)
            if with_count:
                @pl.when(j >= nb)
                def _():
                    pltpu.make_async_copy(
                        ones_v, cnt_sh.at[dst_v.at[j]], csem.at[b]).wait()
            pltpu.make_async_copy(
                tbl_hbm.at[src_v.at[j]], msg_v.at[b], gsem.at[b]).wait()
            pltpu.async_copy(msg_v.at[b], acc_sh.at[dst_v.at[j]], ssem.at[b],
                             add=True)
            if with_count:
                pltpu.async_copy(ones_v, cnt_sh.at[dst_v.at[j]], csem.at[b],
                                 add=True)

            @pl.when(j + nb < n_chunks)
            def _():
                pltpu.make_async_copy(
                    msg_v.at[b], acc_sh.at[dst_v.at[j]], ssem.at[b]).wait()
                gather(j + nb, b)

            return carry

        lax.fori_loop(0, n_chunks, step, 0)
        for b in range(nb):
            pltpu.make_async_copy(
                msg_v.at[b], acc_sh.at[dst_v.at[0]], ssem.at[b]).wait()
            if with_count:
                pltpu.make_async_copy(
                    ones_v, cnt_sh.at[dst_v.at[0]], csem.at[b]).wait()
        plsc.subcore_barrier()
        pltpu.sync_copy(acc_sh.at[rows], acc_out.at[c, rows])
        if with_count:
            pltpu.sync_copy(cnt_sh.at[rows], cnt_out.at[c, rows])

    return pl.kernel(
        body, out_type=out_type, mesh=mesh, scratch_types=scratch,
        compiler_params=pltpu.CompilerParams(use_tc_tiling_on_sc=False))


def _tc_proj(x_pad, Wl, Wr, n_pad, in_ch, out_w):
    grid = (n_pad // BM,)
    return pl.pallas_call(
        _proj_body,
        grid=grid,
        in_specs=[
            pl.BlockSpec((BM, in_ch), lambda i: (i, 0)),
            pl.BlockSpec((in_ch, out_w), lambda i: (0, 0)),
            pl.BlockSpec((in_ch, out_w), lambda i: (0, 0)),
        ],
        out_specs=[
            pl.BlockSpec((BM, out_w), lambda i: (i, 0)),
            pl.BlockSpec((BM, out_w), lambda i: (i, 0)),
        ],
        out_shape=[
            jax.ShapeDtypeStruct((n_pad, out_w), jnp.float32),
            jax.ShapeDtypeStruct((n_pad, out_w), jnp.float32),
        ],
    )(x_pad, Wl, Wr)


def _tc_mid(acc, cnt, xr, bl, Wl2, Wr2, n_pad, hid, out_ch):
    grid = (n_pad // BM,)
    return pl.pallas_call(
        _mid_body,
        grid=grid,
        in_specs=[
            pl.BlockSpec((N_CORES, BM, hid), lambda i: (0, i, 0)),
            pl.BlockSpec((N_CORES, BM, LANES), lambda i: (0, i, 0)),
            pl.BlockSpec((BM, hid), lambda i: (i, 0)),
            pl.BlockSpec((1, hid), lambda i: (0, 0)),
            pl.BlockSpec((hid, out_ch), lambda i: (0, 0)),
            pl.BlockSpec((hid, out_ch), lambda i: (0, 0)),
        ],
        out_specs=[
            pl.BlockSpec((BM, out_ch), lambda i: (i, 0)),
            pl.BlockSpec((BM, out_ch), lambda i: (i, 0)),
        ],
        out_shape=[
            jax.ShapeDtypeStruct((n_pad, out_ch), jnp.float32),
            jax.ShapeDtypeStruct((n_pad, out_ch), jnp.float32),
        ],
    )(acc, cnt, xr, bl, Wl2, Wr2)


def _tc_out(acc, cnt, hr, bl, n_pad, out_ch):
    grid = (n_pad // BM,)
    return pl.pallas_call(
        _out_body,
        grid=grid,
        in_specs=[
            pl.BlockSpec((N_CORES, BM, out_ch), lambda i: (0, i, 0)),
            pl.BlockSpec((N_CORES, BM, LANES), lambda i: (0, i, 0)),
            pl.BlockSpec((BM, out_ch), lambda i: (i, 0)),
            pl.BlockSpec((1, out_ch), lambda i: (0, 0)),
        ],
        out_specs=pl.BlockSpec((BM, out_ch), lambda i: (i, 0)),
        out_shape=jax.ShapeDtypeStruct((n_pad, out_ch), jnp.float32),
    )(acc, cnt, hr, bl)


def kernel(x, edge_index, Wl1, bl1, Wr1, Wl2, bl2, Wr2):
    n, in_ch = x.shape
    hid = Wl1.shape[1]
    out_ch = Wl2.shape[1]
    e = edge_index.shape[1]

    n_pad = -(-n // (N_SUBCORES * BM // 8)) * (N_SUBCORES * BM // 8)
    n_pad = max(n_pad, N_SUBCORES * 8)
    per_tile = -(-e // (N_TILES * CHUNK)) * CHUNK
    n_chunks = per_tile // CHUNK
    e_pad = N_TILES * per_tile

    src = edge_index[0].astype(jnp.int32)
    dst = edge_index[1].astype(jnp.int32)
    pad_node = jnp.int32(n_pad - 1)
    src = jnp.concatenate([src, jnp.full((e_pad - e,), pad_node)])
    dst = jnp.concatenate([dst, jnp.full((e_pad - e,), pad_node)])
    src = src.reshape(N_TILES, n_chunks, CHUNK)
    dst = dst.reshape(N_TILES, n_chunks, CHUNK)

    x_pad = jnp.pad(x, ((0, n_pad - n), (0, 0)))

    zeros_hid = jnp.zeros((n_pad, hid), jnp.float32)
    zeros_cnt = jnp.zeros((n_pad, LANES), jnp.float32)
    zeros_out = jnp.zeros((n_pad, out_ch), jnp.float32)
    ones_rows = jnp.ones((CHUNK, LANES), jnp.float32)

    y1, xr1 = _tc_proj(x_pad, Wl1, Wr1, n_pad, in_ch, hid)
    sc1 = _make_sc_segment_sum(n_pad, hid, n_chunks, with_count=True)
    acc1, cnt = sc1(src, dst, y1, zeros_hid, zeros_cnt, ones_rows)
    y2, hr2 = _tc_mid(acc1, cnt, xr1, bl1.reshape(1, hid), Wl2, Wr2,
                      n_pad, hid, out_ch)

    sc2 = _make_sc_segment_sum(n_pad, out_ch, n_chunks, with_count=False)
    (acc2,) = sc2(src, dst, y2, zeros_out)
    out = _tc_out(acc2, cnt, hr2, bl2.reshape(1, out_ch), n_pad, out_ch)
    return out[:n]
